# Initial kernel scaffold; baseline (speedup 1.0000x reference)
import jax
import jax.numpy as jnp
from jax.experimental import pallas as pl


def kernel(input_ids, word_embeddings):
    raise NotImplementedError("write your pallas kernel here")


# speedup vs baseline: 3.6815x; 3.6815x over previous
import functools

import jax
import jax.numpy as jnp
from jax import lax
from jax.experimental import pallas as pl
from jax.experimental.pallas import tpu as pltpu
from jax.experimental.pallas import tpu_sc as plsc


def _make_gather(V, D, B):
    info = plsc.get_sparse_core_info()
    NC, NS = info.num_cores, info.num_subcores
    NW = NC * NS
    assert B % NW == 0
    b_per_w = B // NW
    C = 128
    assert b_per_w % C == 0
    steps = b_per_w // C

    mesh = plsc.VectorSubcoreMesh(core_axis_name="c", subcore_axis_name="s")

    @functools.partial(
        pl.kernel,
        mesh=mesh,
        out_type=jax.ShapeDtypeStruct((B, D), jnp.float32),
        scratch_types=[
            pltpu.VMEM((b_per_w,), jnp.int32),
            pltpu.VMEM((C, D), jnp.float32),
            pltpu.SemaphoreType.DMA,
        ],
    )
    def k(table_hbm, idx_hbm, out_hbm, idx_v, rows_v, gsem):
        wid = lax.axis_index("s") * NC + lax.axis_index("c")
        base = wid * b_per_w
        pltpu.sync_copy(idx_hbm.at[pl.ds(base, b_per_w)], idx_v)

        def step(g, carry):
            pltpu.async_copy(
                table_hbm.at[idx_v.at[pl.ds(g * C, C)]], rows_v, gsem
            ).wait()
            pltpu.sync_copy(rows_v, out_hbm.at[pl.ds(base + g * C, C)])
            return carry

        lax.fori_loop(0, steps, step, 0)

    return k


_GATHER_CACHE = {}


def kernel(input_ids, word_embeddings):
    V, D = word_embeddings.shape
    shape = input_ids.shape
    B = 1
    for s in shape:
        B *= s
    key = (V, D, B)
    if key not in _GATHER_CACHE:
        _GATHER_CACHE[key] = _make_gather(V, D, B)
    flat = input_ids.reshape(B).astype(jnp.int32)
    out = _GATHER_CACHE[key](word_embeddings, flat)
    return out.reshape(shape + (D,))


# speedup vs baseline: 4.6570x; 1.2650x over previous
import functools

import jax
import jax.numpy as jnp
from jax import lax
from jax.experimental import pallas as pl
from jax.experimental.pallas import tpu as pltpu
from jax.experimental.pallas import tpu_sc as plsc


def _make_gather(V, D, B):
    info = plsc.get_sparse_core_info()
    NC, NS = info.num_cores, info.num_subcores
    NW = NC * NS
    assert B % NW == 0
    b_per_w = B // NW
    C = 128
    assert b_per_w % C == 0
    steps = b_per_w // C

    mesh = plsc.VectorSubcoreMesh(core_axis_name="c", subcore_axis_name="s")

    assert steps % 2 == 0

    @functools.partial(
        pl.kernel,
        mesh=mesh,
        out_type=jax.ShapeDtypeStruct((B, D), jnp.float32),
        scratch_types=[
            pltpu.VMEM((b_per_w,), jnp.int32),
            pltpu.VMEM((C, D), jnp.float32),
            pltpu.VMEM((C, D), jnp.float32),
            pltpu.SemaphoreType.DMA,
            pltpu.SemaphoreType.DMA,
            pltpu.SemaphoreType.DMA,
            pltpu.SemaphoreType.DMA,
        ],
    )
    def k(table_hbm, idx_hbm, out_hbm, idx_v, rows0, rows1, g0s, g1s, o0s, o1s):
        wid = lax.axis_index("s") * NC + lax.axis_index("c")
        base = wid * b_per_w
        pltpu.sync_copy(idx_hbm.at[pl.ds(base, b_per_w)], idx_v)

        def gather(g, buf, sem):
            return pltpu.make_async_copy(
                table_hbm.at[idx_v.at[pl.ds(g * C, C)]], buf, sem
            )

        def put(g, buf, sem):
            return pltpu.make_async_copy(
                buf, out_hbm.at[pl.ds(base + g * C, C)], sem
            )

        gather(0, rows0, g0s).start()
        gather(1, rows1, g1s).start()

        def step(i, carry):
            a, b = 2 * i, 2 * i + 1
            gather(a, rows0, g0s).wait()
            put(a, rows0, o0s).start()
            gather(b, rows1, g1s).wait()
            put(b, rows1, o1s).start()

            @pl.when(a + 2 < steps)
            def _refill():
                put(a, rows0, o0s).wait()
                gather(a + 2, rows0, g0s).start()
                put(b, rows1, o1s).wait()
                gather(b + 2, rows1, g1s).start()

            return carry

        lax.fori_loop(0, steps // 2, step, 0)
        put(steps - 2, rows0, o0s).wait()
        put(steps - 1, rows1, o1s).wait()

    return k


_GATHER_CACHE = {}


def kernel(input_ids, word_embeddings):
    V, D = word_embeddings.shape
    shape = input_ids.shape
    B = 1
    for s in shape:
        B *= s
    key = (V, D, B)
    if key not in _GATHER_CACHE:
        _GATHER_CACHE[key] = _make_gather(V, D, B)
    flat = input_ids.reshape(B).astype(jnp.int32)
    out = _GATHER_CACHE[key](word_embeddings, flat)
    return out.reshape(shape + (D,))


# speedup vs baseline: 4.9547x; 1.0639x over previous
import functools

import jax
import jax.numpy as jnp
from jax import lax
from jax.experimental import pallas as pl
from jax.experimental.pallas import tpu as pltpu
from jax.experimental.pallas import tpu_sc as plsc


def _make_gather(V, D, B):
    info = plsc.get_sparse_core_info()
    NC, NS = info.num_cores, info.num_subcores
    NW = NC * NS
    assert B % NW == 0
    b_per_w = B // NW
    C = 128
    assert b_per_w % C == 0
    steps = b_per_w // C

    mesh = plsc.VectorSubcoreMesh(core_axis_name="c", subcore_axis_name="s")

    NBUF = 5
    assert steps % NBUF == 0
    rounds = steps // NBUF

    @functools.partial(
        pl.kernel,
        mesh=mesh,
        out_type=jax.ShapeDtypeStruct((B, D), jnp.float32),
        scratch_types=[
            pltpu.VMEM((b_per_w,), jnp.int32),
            pltpu.VMEM((NBUF, C, D), jnp.float32),
            [pltpu.SemaphoreType.DMA] * NBUF,
            [pltpu.SemaphoreType.DMA] * NBUF,
        ],
    )
    def k(table_hbm, idx_hbm, out_hbm, idx_v, rows, gsems, osems):
        wid = lax.axis_index("s") * NC + lax.axis_index("c")
        base = wid * b_per_w
        pltpu.sync_copy(idx_hbm.at[pl.ds(base, b_per_w)], idx_v)

        def gather(g, b):
            return pltpu.make_async_copy(
                table_hbm.at[idx_v.at[pl.ds(g * C, C)]], rows.at[b], gsems[b]
            )

        def put(g, b):
            return pltpu.make_async_copy(
                rows.at[b], out_hbm.at[pl.ds(base + g * C, C)], osems[b]
            )

        for b in range(NBUF):
            gather(b, b).start()

        def step(i, carry):
            g0 = i * NBUF
            for b in range(NBUF):
                gather(g0 + b, b).wait()
                put(g0 + b, b).start()
            for b in range(NBUF):
                put(g0 + b, b).wait()
                gather(g0 + NBUF + b, b).start()
            return carry

        lax.fori_loop(0, rounds - 1, step, 0)

        g0 = (rounds - 1) * NBUF
        for b in range(NBUF):
            gather(g0 + b, b).wait()
            put(g0 + b, b).start()
        for b in range(NBUF):
            put(g0 + b, b).wait()

    return k


_GATHER_CACHE = {}


def kernel(input_ids, word_embeddings):
    V, D = word_embeddings.shape
    shape = input_ids.shape
    B = 1
    for s in shape:
        B *= s
    key = (V, D, B)
    if key not in _GATHER_CACHE:
        _GATHER_CACHE[key] = _make_gather(V, D, B)
    flat = input_ids.reshape(B).astype(jnp.int32)
    out = _GATHER_CACHE[key](word_embeddings, flat)
    return out.reshape(shape + (D,))


# speedup vs baseline: 5.0759x; 1.0245x over previous
import functools

import jax
import jax.numpy as jnp
from jax import lax
from jax.experimental import pallas as pl
from jax.experimental.pallas import tpu as pltpu
from jax.experimental.pallas import tpu_sc as plsc


def _make_gather(V, D, B):
    info = plsc.get_sparse_core_info()
    NC, NS = info.num_cores, info.num_subcores
    NW = NC * NS
    assert B % NW == 0
    b_per_w = B // NW
    C = 64
    assert b_per_w % C == 0
    steps = b_per_w // C

    mesh = plsc.VectorSubcoreMesh(core_axis_name="c", subcore_axis_name="s")

    NBUF = 10
    assert steps % NBUF == 0
    rounds = steps // NBUF

    @functools.partial(
        pl.kernel,
        mesh=mesh,
        out_type=jax.ShapeDtypeStruct((B, D), jnp.float32),
        scratch_types=[
            pltpu.VMEM((b_per_w,), jnp.int32),
            pltpu.VMEM((NBUF, C, D), jnp.float32),
            [pltpu.SemaphoreType.DMA] * NBUF,
            [pltpu.SemaphoreType.DMA] * NBUF,
        ],
    )
    def k(table_hbm, idx_hbm, out_hbm, idx_v, rows, gsems, osems):
        wid = lax.axis_index("s") * NC + lax.axis_index("c")
        base = wid * b_per_w
        pltpu.sync_copy(idx_hbm.at[pl.ds(base, b_per_w)], idx_v)

        def gather(g, b):
            return pltpu.make_async_copy(
                table_hbm.at[idx_v.at[pl.ds(g * C, C)]], rows.at[b], gsems[b]
            )

        def put(g, b):
            return pltpu.make_async_copy(
                rows.at[b], out_hbm.at[pl.ds(base + g * C, C)], osems[b]
            )

        for b in range(NBUF):
            gather(b, b).start()

        def step(i, carry):
            g0 = i * NBUF
            for b in range(NBUF):
                gather(g0 + b, b).wait()
                put(g0 + b, b).start()
            for b in range(NBUF):
                put(g0 + b, b).wait()
                gather(g0 + NBUF + b, b).start()
            return carry

        lax.fori_loop(0, rounds - 1, step, 0)

        g0 = (rounds - 1) * NBUF
        for b in range(NBUF):
            gather(g0 + b, b).wait()
            put(g0 + b, b).start()
        for b in range(NBUF):
            put(g0 + b, b).wait()

    return k


_GATHER_CACHE = {}


def kernel(input_ids, word_embeddings):
    V, D = word_embeddings.shape
    shape = input_ids.shape
    B = 1
    for s in shape:
        B *= s
    key = (V, D, B)
    if key not in _GATHER_CACHE:
        _GATHER_CACHE[key] = _make_gather(V, D, B)
    flat = input_ids.reshape(B).astype(jnp.int32)
    out = _GATHER_CACHE[key](word_embeddings, flat)
    return out.reshape(shape + (D,))
